# baseline (speedup 1.0000x reference)
import functools

import jax
import jax.numpy as jnp
from jax import lax
from jax.experimental import pallas as pl
from jax.experimental.pallas import tpu as pltpu
from jax.experimental.pallas import tpu_sc as plsc

_N = 10000
_E = 320000
_H = 128
_TPC = 16
_NBLK = 32
_EPB = _E // _NBLK
_CH = 80
_NCH = _EPB // _CH
_HALF = _N // 2
_ACC = 5024

_mesh = plsc.VectorSubcoreMesh(core_axis_name="c", subcore_axis_name="s")
_i32 = jnp.int32
_f32 = jnp.float32


def _stage_local_dst(dst_hbm, odst2, cid, sid, blk):
    pltpu.sync_copy(dst_hbm.at[blk], odst2)
    lo_v = jnp.broadcast_to(cid * _HALF, (16,)).astype(_i32)
    hu = jnp.full((16,), _HALF, jnp.uint32)
    iota = lax.broadcasted_iota(_i32, (16,), 0)
    trash_v = (jnp.broadcast_to(5000 + 8 * (sid % 2), (16,)).astype(_i32)
               + (iota & 7))

    def clamp_body(r, carry):
        for k in range(_CH // 16):
            d = odst2[r, pl.ds(k * 16, 16)]
            dl = d - lo_v
            m = plsc.bitcast(dl, jnp.uint32) < hu
            odst2[r, pl.ds(k * 16, 16)] = jnp.where(m, dl, trash_v)
        return carry

    lax.fori_loop(0, _NCH, clamp_body, 0)


def _stage_conv_idx(src_hbm, dst_hbm, src_v, dst_v, cid, sid, blk):
    pltpu.sync_copy(src_hbm.at[blk], src_v)
    pltpu.sync_copy(dst_hbm.at[blk], dst_v)
    lo_v = jnp.broadcast_to(cid * _HALF, (16,)).astype(_i32)
    hu = jnp.full((16,), _HALF, jnp.uint32)
    iota = lax.broadcasted_iota(_i32, (16,), 0)
    zsrc = (jnp.broadcast_to(_N + 7 * sid, (16,)).astype(_i32)
            + iota * 125)
    zdst = (jnp.broadcast_to(4 * sid, (16,)).astype(_i32) + iota * 64)

    def clamp_body(r, carry):
        for k in range(_CH // 16):
            sv = src_v[r, pl.ds(k * 16, 16)]
            d = dst_v[r, pl.ds(k * 16, 16)]
            dl = d - lo_v
            m = plsc.bitcast(dl, jnp.uint32) < hu
            src_v[r, pl.ds(k * 16, 16)] = jnp.where(m, sv, zsrc)
            dst_v[r, pl.ds(k * 16, 16)] = jnp.where(m, dl, zdst)
        return carry

    lax.fori_loop(0, _NCH, clamp_body, 0)


@functools.partial(
    pl.kernel,
    out_type=jax.ShapeDtypeStruct((_N,), _f32),
    mesh=_mesh,
    scratch_types=[
        pltpu.VMEM((_NCH, _CH), _i32),
        pltpu.VMEM((_CH,), _f32),
        pltpu.VMEM((1024,), _f32),
        pltpu.VMEM_SHARED((_ACC,), _f32),
    ],
)
def _prep_sc(dst_hbm, deg_hbm, odst2, ones_v, zslab, dacc):
    cid = lax.axis_index("c")
    sid = lax.axis_index("s")
    for k in range(_CH // 16):
        ones_v[pl.ds(k * 16, 16)] = jnp.ones((16,), _f32)
    for k in range(1024 // 16):
        zslab[pl.ds(k * 16, 16)] = jnp.zeros((16,), _f32)
    @pl.when(sid < 4)
    def _():
        pltpu.sync_copy(zslab, dacc.at[pl.ds(sid * 1024, 1024)])
    @pl.when(sid == 4)
    def _():
        pltpu.sync_copy(zslab.at[pl.ds(0, 928)], dacc.at[pl.ds(4096, 928)])
    plsc.subcore_barrier()

    def deg_body(j, carry):
        pltpu.sync_copy(ones_v, dacc.at[odst2.at[j]], add=True)
        return carry

    for p in range(2):
        _stage_local_dst(dst_hbm, odst2, cid, sid, 2 * sid + p)
        lax.fori_loop(0, _NCH, deg_body, 0)
    plsc.subcore_barrier()
    @pl.when(sid < 5)
    def _():
        pltpu.sync_copy(dacc.at[pl.ds(sid * 1000, 1000)],
                        zslab.at[pl.ds(0, 1000)])
        pltpu.sync_copy(zslab.at[pl.ds(0, 1000)],
                        deg_hbm.at[pl.ds(cid * _HALF + sid * 1000, 1000)])


@functools.partial(
    pl.kernel,
    out_type=jax.ShapeDtypeStruct((_N, _H), _f32),
    mesh=_mesh,
    scratch_types=[
        pltpu.VMEM((_NCH, _CH), _i32),
        pltpu.VMEM((_NCH, _CH), _i32),
        pltpu.VMEM((_CH, _H), _f32),
        pltpu.VMEM((_CH, _H), _f32),
        pltpu.VMEM((_CH, _H), _f32),
        pltpu.VMEM((_CH, _H), _f32),
        pltpu.VMEM((32, _H), _f32),
        pltpu.VMEM((64, _H), _f32),
        pltpu.VMEM_SHARED((_ACC, _H), _f32),
        pltpu.SemaphoreType.DMA,
        pltpu.SemaphoreType.DMA,
        pltpu.SemaphoreType.DMA,
        pltpu.SemaphoreType.DMA,
    ],
)
def _conv_sc(a_hbm, src_hbm, dst_hbm, out_hbm, src_v, dst_v,
             buf0, buf1, buf2, buf3, zbuf, wbuf, acc,
             sem0, sem1, sem2, sem3):
    cid = lax.axis_index("c")
    sid = lax.axis_index("s")
    for r in range(32):
        for k in range(_H // 16):
            zbuf[r, pl.ds(k * 16, 16)] = jnp.zeros((16,), _f32)
    @pl.when(sid < 15)
    def _():
        for t in range(10):
            pltpu.sync_copy(zbuf, acc.at[pl.ds(sid * 320 + t * 32, 32)])
    @pl.when(sid == 15)
    def _():
        for t in range(7):
            pltpu.sync_copy(zbuf, acc.at[pl.ds(4800 + t * 32, 32)])

    bufs = (buf0, buf1, buf2, buf3)
    sems = (sem0, sem1, sem2, sem3)

    def quad(q, carry):
        j = 4 * q
        ds = [pltpu.async_copy(a_hbm.at[src_v.at[j + t]], bufs[t], sems[t])
              for t in range(4)]
        for t in range(4):
            ds[t].wait()
            pltpu.sync_copy(bufs[t], acc.at[dst_v.at[j + t]], add=True)
        return carry

    first = True
    for p in range(2):
        blk = 2 * sid + p
        _stage_conv_idx(src_hbm, dst_hbm, src_v, dst_v, cid, sid, blk)
        if first:
            plsc.subcore_barrier()
            first = False
        lax.fori_loop(0, _NCH // 4, quad, 0)
        last = _NCH - 1
        dl = pltpu.async_copy(a_hbm.at[src_v.at[last]], buf0, sem0)
        dl.wait()
        pltpu.sync_copy(buf0, acc.at[dst_v.at[last]], add=True)
    plsc.subcore_barrier()
    @pl.when(sid < 9)
    def _():
        for t in range(8):
            base = sid * 512 + t * 64
            pltpu.sync_copy(acc.at[pl.ds(base, 64)], wbuf)
            pltpu.sync_copy(wbuf, out_hbm.at[pl.ds(cid * _HALF + base, 64)])
    @pl.when(sid == 9)
    def _():
        for t in range(6):
            base = 4608 + t * 64
            pltpu.sync_copy(acc.at[pl.ds(base, 64)], wbuf)
            pltpu.sync_copy(wbuf, out_hbm.at[pl.ds(cid * _HALF + base, 64)])
        pltpu.sync_copy(acc.at[pl.ds(4992, 8)], wbuf.at[pl.ds(0, 8)])
        pltpu.sync_copy(wbuf.at[pl.ds(0, 8)],
                        out_hbm.at[pl.ds(cid * _HALF + 4992, 8)])


_R = 2000


def _mm(a, b):
    return lax.dot_general(a, b, (((1,), (1,)), ((), ())),
                           preferred_element_type=_f32)


def _bnorm(h, g, b, m, v):
    return (h - m) * lax.rsqrt(v + 1e-5) * g + b


def _prep_mm_body(x, w1, hw_o):
    hw_o[...] = _mm(x[...], w1[...])


def _prep_scale_body(d, hw, a_o, dis_o):
    i = pl.program_id(0)
    dis = lax.rsqrt(d[...] + 1.0)
    val = hw[...] * dis
    a_o[...] = jnp.where(i == 5, jnp.zeros_like(val), val)
    dis_o[...] = dis


def _mid_body(s, a, dis, bias, g, b, m, v, w, a2_o):
    i = pl.program_id(0)
    t = dis[...] * (s[...] + a[...]) + bias[...]
    h = jnp.maximum(_bnorm(t, g[...], b[...], m[...], v[...]), 0.0)
    val = _mm(h, w[...]) * dis[...]
    a2_o[...] = jnp.where(i == 5, jnp.zeros_like(val), val)


def _tail_body(s, a, dis, b2, g2, bb2, m2, v2, fc1w, fc1b,
               g3, bb3, m3, v3, fc2w, fc2b, outw, outb, h1_o, pred_o):
    t = dis[...] * (s[...] + a[...]) + b2[...]
    h = jnp.maximum(_bnorm(t, g2[...], bb2[...], m2[...], v2[...]), 0.0)
    h = _mm(h, fc1w[...]) + fc1b[...]
    h = jnp.maximum(_bnorm(h, g3[...], bb3[...], m3[...], v3[...]), 0.0)
    h = _mm(h, fc2w[...]) + fc2b[...]
    h1 = jnp.maximum(h, 0.0)
    h1_o[...] = h1
    z = jnp.sum(h1 * outw[...], axis=1, keepdims=True) + outb[...]
    pred_o[...] = jnp.log(1.0 + jnp.exp(-jnp.abs(z))) + jnp.maximum(z, 0.0)


def _col(i):
    return (i, 0)


_bsR1 = pl.BlockSpec((_R, 1), _col)
_bsRH = pl.BlockSpec((_R, _H), _col)
_bsRHm = pl.BlockSpec((_R, _H), lambda i: (i % 5, 0))
_bsHH = pl.BlockSpec((_H, _H), lambda i: (0, 0))
_bs1H = pl.BlockSpec((1, _H), lambda i: (0, 0))
_bs11 = pl.BlockSpec((1, 1), lambda i: (0, 0))
_GRID = (_N // _R,)


def kernel(x, edge_index, W1, b1, W2, b2, fc1_W, fc1_b, fc2_W, fc2_b,
           out_W, out_b, bn1_g, bn1_b, bn1_m, bn1_v, bn2_g, bn2_b, bn2_m,
           bn2_v, bn3_g, bn3_b, bn3_m, bn3_v):
    src = edge_index[0].reshape(_NBLK, _NCH, _CH)
    dst = edge_index[1].reshape(_NBLK, _NCH, _CH)
    deg = _prep_sc(dst)

    hw1 = pl.pallas_call(
        _prep_mm_body,
        grid=_GRID,
        in_specs=[_bsRH, _bsHH],
        out_specs=_bsRH,
        out_shape=jax.ShapeDtypeStruct((_N, _H), _f32),
    )(x, W1)

    deg_b = jnp.broadcast_to(deg.reshape(_N, 1), (_N, _H))
    a1, dis = pl.pallas_call(
        _prep_scale_body,
        grid=(6,),
        in_specs=[_bsRHm, _bsRHm],
        out_specs=[_bsRH, _bsRHm],
        out_shape=[jax.ShapeDtypeStruct((_N + 2000, _H), _f32),
                   jax.ShapeDtypeStruct((_N, _H), _f32)],
    )(deg_b, hw1)

    s1 = _conv_sc(a1, src, dst)

    a2 = pl.pallas_call(
        _mid_body,
        grid=(6,),
        in_specs=[_bsRHm, _bsRHm, _bsRHm, _bs1H, _bs1H, _bs1H, _bs1H, _bs1H,
                  _bsHH],
        out_specs=_bsRH,
        out_shape=jax.ShapeDtypeStruct((_N + 2000, _H), _f32),
    )(s1, a1, dis, b1.reshape(1, _H), bn1_g.reshape(1, _H),
      bn1_b.reshape(1, _H), bn1_m.reshape(1, _H), bn1_v.reshape(1, _H), W2)

    s2 = _conv_sc(a2, src, dst)

    h1, pred = pl.pallas_call(
        _tail_body,
        grid=_GRID,
        in_specs=[_bsRH, _bsRH, _bsRH, _bs1H, _bs1H, _bs1H, _bs1H, _bs1H,
                  _bsHH, _bs1H, _bs1H, _bs1H, _bs1H, _bs1H, _bsHH, _bs1H,
                  _bs1H, _bs11],
        out_specs=[_bsRH, _bsR1],
        out_shape=[jax.ShapeDtypeStruct((_N, _H), _f32),
                   jax.ShapeDtypeStruct((_N, 1), _f32)],
    )(s2, a2, dis, b2.reshape(1, _H), bn2_g.reshape(1, _H),
      bn2_b.reshape(1, _H), bn2_m.reshape(1, _H), bn2_v.reshape(1, _H),
      fc1_W, fc1_b.reshape(1, _H), bn3_g.reshape(1, _H), bn3_b.reshape(1, _H),
      bn3_m.reshape(1, _H), bn3_v.reshape(1, _H), fc2_W, fc2_b.reshape(1, _H),
      out_W, out_b.reshape(1, 1))

    return (pred.reshape(-1), h1)


# speedup vs baseline: 1.2624x; 1.2624x over previous
import functools

import jax
import jax.numpy as jnp
from jax import lax
from jax.experimental import pallas as pl
from jax.experimental.pallas import tpu as pltpu
from jax.experimental.pallas import tpu_sc as plsc

_N = 10000
_E = 320000
_H = 128
_TPC = 16
_NBLK = 32
_EPB = _E // _NBLK
_CH = 80
_NCH = _EPB // _CH
_HALF = _N // 2
_ACC = 5024

_mesh = plsc.VectorSubcoreMesh(core_axis_name="c", subcore_axis_name="s")
_i32 = jnp.int32
_f32 = jnp.float32


def _stage_local_dst(dst_hbm, odst2, cid, sid, blk):
    pltpu.sync_copy(dst_hbm.at[blk], odst2)
    lo_v = jnp.broadcast_to(cid * _HALF, (16,)).astype(_i32)
    hu = jnp.full((16,), _HALF, jnp.uint32)
    iota = lax.broadcasted_iota(_i32, (16,), 0)
    trash_v = (jnp.broadcast_to(5000 + 8 * (sid % 2), (16,)).astype(_i32)
               + (iota & 7))

    def clamp_body(r, carry):
        for k in range(_CH // 16):
            d = odst2[r, pl.ds(k * 16, 16)]
            dl = d - lo_v
            m = plsc.bitcast(dl, jnp.uint32) < hu
            odst2[r, pl.ds(k * 16, 16)] = jnp.where(m, dl, trash_v)
        return carry

    lax.fori_loop(0, _NCH, clamp_body, 0)


@functools.partial(
    pl.kernel,
    out_type=jax.ShapeDtypeStruct((_N,), _f32),
    mesh=_mesh,
    scratch_types=[
        pltpu.VMEM((_NCH, _CH), _i32),
        pltpu.VMEM((_CH,), _f32),
        pltpu.VMEM((1024,), _f32),
        pltpu.VMEM_SHARED((_ACC,), _f32),
    ],
)
def _prep_sc(dst_hbm, deg_hbm, odst2, ones_v, zslab, dacc):
    cid = lax.axis_index("c")
    sid = lax.axis_index("s")
    for k in range(_CH // 16):
        ones_v[pl.ds(k * 16, 16)] = jnp.ones((16,), _f32)
    for k in range(1024 // 16):
        zslab[pl.ds(k * 16, 16)] = jnp.zeros((16,), _f32)
    @pl.when(sid < 4)
    def _():
        pltpu.sync_copy(zslab, dacc.at[pl.ds(sid * 1024, 1024)])
    @pl.when(sid == 4)
    def _():
        pltpu.sync_copy(zslab.at[pl.ds(0, 928)], dacc.at[pl.ds(4096, 928)])
    plsc.subcore_barrier()

    def deg_body(j, carry):
        pltpu.sync_copy(ones_v, dacc.at[odst2.at[j]], add=True)
        return carry

    for p in range(2):
        _stage_local_dst(dst_hbm, odst2, cid, sid, 2 * sid + p)
        lax.fori_loop(0, _NCH, deg_body, 0)
    plsc.subcore_barrier()
    @pl.when(sid < 5)
    def _():
        pltpu.sync_copy(dacc.at[pl.ds(sid * 1000, 1000)],
                        zslab.at[pl.ds(0, 1000)])
        pltpu.sync_copy(zslab.at[pl.ds(0, 1000)],
                        deg_hbm.at[pl.ds(cid * _HALF + sid * 1000, 1000)])


@functools.partial(
    pl.kernel,
    out_type=jax.ShapeDtypeStruct((_N, _H), _f32),
    mesh=_mesh,
    scratch_types=[
        pltpu.VMEM((_NCH, _CH), _i32),
        pltpu.VMEM((_NCH, _CH), _i32),
        pltpu.VMEM((_CH, _H), _f32),
        pltpu.VMEM((_CH, _H), _f32),
        pltpu.VMEM((_CH, _H), _f32),
        pltpu.VMEM((_CH, _H), _f32),
        pltpu.VMEM((_CH, _H), _f32),
        pltpu.VMEM((16, _H), _f32),
        pltpu.VMEM((32, _H), _f32),
        pltpu.VMEM_SHARED((_ACC, _H), _f32),
        pltpu.SemaphoreType.DMA,
        pltpu.SemaphoreType.DMA,
        pltpu.SemaphoreType.DMA,
        pltpu.SemaphoreType.DMA,
        pltpu.SemaphoreType.DMA,
    ],
)
def _conv_sc(a_hbm, src_hbm, dst_hbm, out_hbm, src_v, dst_v,
             buf0, buf1, buf2, buf3, buf4, zbuf, wbuf, acc,
             sem0, sem1, sem2, sem3, sem4):
    cid = lax.axis_index("c")
    sid = lax.axis_index("s")
    for r in range(16):
        for k in range(_H // 16):
            zbuf[r, pl.ds(k * 16, 16)] = jnp.zeros((16,), _f32)
    @pl.when(sid < 15)
    def _():
        for t in range(20):
            pltpu.sync_copy(zbuf, acc.at[pl.ds(sid * 320 + t * 16, 16)])
    @pl.when(sid == 15)
    def _():
        for t in range(14):
            pltpu.sync_copy(zbuf, acc.at[pl.ds(4800 + t * 16, 16)])

    bufs = (buf0, buf1, buf2, buf3, buf4)
    sems = (sem0, sem1, sem2, sem3, sem4)

    def quint(q, carry):
        j = 5 * q
        ds = [pltpu.async_copy(a_hbm.at[src_v.at[j + t]], bufs[t], sems[t])
              for t in range(5)]
        for t in range(5):
            ds[t].wait()
            pltpu.sync_copy(bufs[t], acc.at[dst_v.at[j + t]], add=True)
        return carry

    first = True
    for p in range(2):
        blk = 2 * sid + p
        pltpu.sync_copy(src_hbm.at[blk], src_v)
        _stage_local_dst(dst_hbm, dst_v, cid, sid, blk)
        if first:
            plsc.subcore_barrier()
            first = False
        lax.fori_loop(0, _NCH // 5, quint, 0)
    plsc.subcore_barrier()
    @pl.when(sid < 9)
    def _():
        for t in range(16):
            base = sid * 512 + t * 32
            pltpu.sync_copy(acc.at[pl.ds(base, 32)], wbuf)
            pltpu.sync_copy(wbuf, out_hbm.at[pl.ds(cid * _HALF + base, 32)])
    @pl.when(sid == 9)
    def _():
        for t in range(12):
            base = 4608 + t * 32
            pltpu.sync_copy(acc.at[pl.ds(base, 32)], wbuf)
            pltpu.sync_copy(wbuf, out_hbm.at[pl.ds(cid * _HALF + base, 32)])
        pltpu.sync_copy(acc.at[pl.ds(4992, 8)], wbuf.at[pl.ds(0, 8)])
        pltpu.sync_copy(wbuf.at[pl.ds(0, 8)],
                        out_hbm.at[pl.ds(cid * _HALF + 4992, 8)])


_R = 2000


def _mm(a, b):
    return lax.dot_general(a, b, (((1,), (1,)), ((), ())),
                           preferred_element_type=_f32)


def _bnorm(h, g, b, m, v):
    return (h - m) * lax.rsqrt(v + 1e-5) * g + b


def _prep_mm_body(x, w1, hw_o):
    hw_o[...] = _mm(x[...], w1[...])


def _prep_scale_body(d, hw, a_o, dis_o):
    dis = lax.rsqrt(d[...] + 1.0)
    a_o[...] = hw[...] * dis
    dis_o[...] = dis


def _mid_body(s, a, dis, bias, g, b, m, v, w, a2_o):
    t = dis[...] * (s[...] + a[...]) + bias[...]
    h = jnp.maximum(_bnorm(t, g[...], b[...], m[...], v[...]), 0.0)
    a2_o[...] = _mm(h, w[...]) * dis[...]


def _tail_body(s, a, dis, b2, g2, bb2, m2, v2, fc1w, fc1b,
               g3, bb3, m3, v3, fc2w, fc2b, outw, outb, h1_o, pred_o):
    t = dis[...] * (s[...] + a[...]) + b2[...]
    h = jnp.maximum(_bnorm(t, g2[...], bb2[...], m2[...], v2[...]), 0.0)
    h = _mm(h, fc1w[...]) + fc1b[...]
    h = jnp.maximum(_bnorm(h, g3[...], bb3[...], m3[...], v3[...]), 0.0)
    h = _mm(h, fc2w[...]) + fc2b[...]
    h1 = jnp.maximum(h, 0.0)
    h1_o[...] = h1
    z = jnp.sum(h1 * outw[...], axis=1, keepdims=True) + outb[...]
    pred_o[...] = jnp.log(1.0 + jnp.exp(-jnp.abs(z))) + jnp.maximum(z, 0.0)


def _col(i):
    return (i, 0)


_bsR1 = pl.BlockSpec((_R, 1), _col)
_bsRH = pl.BlockSpec((_R, _H), _col)
_bsHH = pl.BlockSpec((_H, _H), lambda i: (0, 0))
_bs1H = pl.BlockSpec((1, _H), lambda i: (0, 0))
_bs11 = pl.BlockSpec((1, 1), lambda i: (0, 0))
_GRID = (_N // _R,)


def kernel(x, edge_index, W1, b1, W2, b2, fc1_W, fc1_b, fc2_W, fc2_b,
           out_W, out_b, bn1_g, bn1_b, bn1_m, bn1_v, bn2_g, bn2_b, bn2_m,
           bn2_v, bn3_g, bn3_b, bn3_m, bn3_v):
    src = edge_index[0].reshape(_NBLK, _NCH, _CH)
    dst = edge_index[1].reshape(_NBLK, _NCH, _CH)
    deg = _prep_sc(dst)

    hw1 = pl.pallas_call(
        _prep_mm_body,
        grid=_GRID,
        in_specs=[_bsRH, _bsHH],
        out_specs=_bsRH,
        out_shape=jax.ShapeDtypeStruct((_N, _H), _f32),
    )(x, W1)

    deg_b = jnp.broadcast_to(deg.reshape(_N, 1), (_N, _H))
    a1, dis = pl.pallas_call(
        _prep_scale_body,
        grid=_GRID,
        in_specs=[_bsRH, _bsRH],
        out_specs=[_bsRH, _bsRH],
        out_shape=[jax.ShapeDtypeStruct((_N, _H), _f32),
                   jax.ShapeDtypeStruct((_N, _H), _f32)],
    )(deg_b, hw1)

    s1 = _conv_sc(a1, src, dst)

    a2 = pl.pallas_call(
        _mid_body,
        grid=_GRID,
        in_specs=[_bsRH, _bsRH, _bsRH, _bs1H, _bs1H, _bs1H, _bs1H, _bs1H,
                  _bsHH],
        out_specs=_bsRH,
        out_shape=jax.ShapeDtypeStruct((_N, _H), _f32),
    )(s1, a1, dis, b1.reshape(1, _H), bn1_g.reshape(1, _H),
      bn1_b.reshape(1, _H), bn1_m.reshape(1, _H), bn1_v.reshape(1, _H), W2)

    s2 = _conv_sc(a2, src, dst)

    h1, pred = pl.pallas_call(
        _tail_body,
        grid=_GRID,
        in_specs=[_bsRH, _bsRH, _bsRH, _bs1H, _bs1H, _bs1H, _bs1H, _bs1H,
                  _bsHH, _bs1H, _bs1H, _bs1H, _bs1H, _bs1H, _bsHH, _bs1H,
                  _bs1H, _bs11],
        out_specs=[_bsRH, _bsR1],
        out_shape=[jax.ShapeDtypeStruct((_N, _H), _f32),
                   jax.ShapeDtypeStruct((_N, 1), _f32)],
    )(s2, a2, dis, b2.reshape(1, _H), bn2_g.reshape(1, _H),
      bn2_b.reshape(1, _H), bn2_m.reshape(1, _H), bn2_v.reshape(1, _H),
      fc1_W, fc1_b.reshape(1, _H), bn3_g.reshape(1, _H), bn3_b.reshape(1, _H),
      bn3_m.reshape(1, _H), bn3_v.reshape(1, _H), fc2_W, fc2_b.reshape(1, _H),
      out_W, out_b.reshape(1, 1))

    return (pred.reshape(-1), h1)
